# baseline (speedup 1.0000x reference)
import functools

import jax
import jax.numpy as jnp
from jax import lax
from jax.experimental import pallas as pl
from jax.experimental.pallas import tpu as pltpu
from jax.experimental.pallas import tpu_sc as plsc

N = 10000
E = 320000
D = 128
H = 128
P = 16

NC = 2
NS = 16
NW = NC * NS
EPW = E // NW
CB = 80
NCHUNK = EPW // CB
NPAD = 10240
NPW = NPAD // NS

BN = 2000
BE = 8000
NSLICE = 1
DW = 128


def _silu(v):
    return v * (1.0 / (1.0 + jnp.exp(-v)))


def _precompute_body(x_ref, wa_ref, wb_ref, b1e_ref, wna_ref,
                     b1n_ref, xa_ref, xb_ref, xc_ref):
    xv = x_ref[...]
    xa_ref[...] = jnp.dot(xv, wa_ref[...],
                          preferred_element_type=jnp.float32) + b1e_ref[...]
    xb_ref[...] = jnp.dot(xv, wb_ref[...], preferred_element_type=jnp.float32)
    xc_ref[...] = jnp.dot(xv, wna_ref[...],
                          preferred_element_type=jnp.float32) + b1n_ref[...]


def _precompute(x, W1e_a, W1e_b, b1e, W1n_a, b1n):
    f32 = jnp.float32
    return pl.pallas_call(
        _precompute_body,
        grid=(N // BN,),
        in_specs=[
            pl.BlockSpec((BN, D), lambda i: (i, 0)),
            pl.BlockSpec((D, H), lambda i: (0, 0)),
            pl.BlockSpec((D, H), lambda i: (0, 0)),
            pl.BlockSpec((1, H), lambda i: (0, 0)),
            pl.BlockSpec((D, H), lambda i: (0, 0)),
            pl.BlockSpec((1, H), lambda i: (0, 0)),
        ],
        out_specs=[
            pl.BlockSpec((BN, H), lambda i: (i, 0)),
            pl.BlockSpec((BN, H), lambda i: (i, 0)),
            pl.BlockSpec((BN, H), lambda i: (i, 0)),
        ],
        out_shape=[
            jax.ShapeDtypeStruct((N, H), f32),
            jax.ShapeDtypeStruct((N, H), f32),
            jax.ShapeDtypeStruct((N, H), f32),
        ],
    )(x, W1e_a, W1e_b, b1e, W1n_a, b1n)


def _make_edge_gather_body(epw, nchunk):
  def _edge_gather_body(row_hbm, col_hbm, xa_hbm, xb_hbm, px_hbm, py_hbm,
                      pz_hbm, xar_hbm, xbc_hbm, rij_hbm,
                      idxr, idxc, bufA0, bufB0, bufA1, bufB1,
                      bufA2, bufB2, bufR, px_v, py_v, pz_v, semG, semW, semR):
    wid = lax.axis_index("s") * NC + lax.axis_index("c")
    base = wid * epw

    pltpu.sync_copy(px_hbm, px_v)
    pltpu.sync_copy(py_hbm, py_v)
    pltpu.sync_copy(pz_hbm, pz_v)
    pltpu.sync_copy(row_hbm.at[pl.ds(base, epw)], idxr)
    pltpu.sync_copy(col_hbm.at[pl.ds(base, epw)], idxc)
    lane = lax.iota(jnp.int32, 16)

    def zr(r, carry):
        bufR[r, pl.ds(0, P)] = jnp.zeros((P,), jnp.float32)
        return carry

    lax.fori_loop(0, CB, zr, 0)

    def issue(ci, bA, bB):
        pltpu.async_copy(xa_hbm.at[idxr.at[pl.ds(ci * CB, CB)]], bA, semG)
        pltpu.async_copy(xb_hbm.at[idxc.at[pl.ds(ci * CB, CB)]], bB, semG)

    def drain_w():
        pltpu.make_async_copy(bufA0, xar_hbm.at[pl.ds(base, CB)], semW).wait()
        pltpu.make_async_copy(bufB0, xbc_hbm.at[pl.ds(base, CB)], semW).wait()

    def drain_r():
        pltpu.make_async_copy(bufR, rij_hbm.at[pl.ds(base, CB)], semR).wait()

    def process(ci, bA, bB):
        @pl.when(ci > 0)
        def _():
            drain_r()

        def sub(k, carry2):
            off = ci * CB + k * 16
            ir = idxr[pl.ds(off, 16)]
            ic = idxc[pl.ds(off, 16)]
            rows = k * 16 + lane
            for c, pv in enumerate((px_v, py_v, pz_v)):
                d = plsc.load_gather(pv, [ir]) - plsc.load_gather(pv, [ic])
                plsc.store_scatter(bufR,
                                   [rows, jnp.full((16,), c, jnp.int32)], d)
            return carry2

        lax.fori_loop(0, CB // 16, sub, 0)
        pltpu.make_async_copy(xa_hbm.at[pl.ds(0, CB)], bA, semG).wait()
        pltpu.make_async_copy(xa_hbm.at[pl.ds(0, CB)], bB, semG).wait()
        cb = base + ci * CB
        pltpu.async_copy(bA, xar_hbm.at[pl.ds(cb, CB)], semW)
        pltpu.async_copy(bB, xbc_hbm.at[pl.ds(cb, CB)], semW)
        pltpu.async_copy(bufR, rij_hbm.at[pl.ds(cb, CB)], semR)

    issue(0, bufA0, bufB0)
    issue(1, bufA1, bufB1)
    issue(2, bufA2, bufB2)

    def triple(i, carry):
        c0 = 3 * i
        sets = ((bufA0, bufB0), (bufA1, bufB1), (bufA2, bufB2))
        for k, (bA, bB) in enumerate(sets):
            process(c0 + k, bA, bB)

            @pl.when(c0 + k + 3 < nchunk)
            def _():
                drain_w()
                issue(c0 + k + 3, bA, bB)

        return carry

    lax.fori_loop(0, nchunk // 3, triple, 0)
    for k in range(nchunk % 3):
        process(nchunk - (nchunk % 3) + k,
                (bufA0, bufA1, bufA2)[k], (bufB0, bufB1, bufB2)[k])
    for _ in range(3):
        drain_w()
    drain_r()
  return _edge_gather_body


def _edge_gather(row, col, xa, xb, px, py, pz, ne):
    f32 = jnp.float32
    epw = ne // NW
    nchunk = epw // CB
    mesh = plsc.VectorSubcoreMesh(core_axis_name="c", subcore_axis_name="s",
                                  num_cores=NC, num_subcores=NS)
    fn = functools.partial(
        pl.kernel, mesh=mesh,
        compiler_params=pltpu.CompilerParams(needs_layout_passes=False),
        out_type=[
            jax.ShapeDtypeStruct((ne, H), f32),
            jax.ShapeDtypeStruct((ne, H), f32),
            jax.ShapeDtypeStruct((ne, P), f32),
        ],
        scratch_types=[
            pltpu.VMEM((epw,), jnp.int32),
            pltpu.VMEM((epw,), jnp.int32),
            pltpu.VMEM((CB, H), f32),
            pltpu.VMEM((CB, H), f32),
            pltpu.VMEM((CB, H), f32),
            pltpu.VMEM((CB, H), f32),
            pltpu.VMEM((CB, H), f32),
            pltpu.VMEM((CB, H), f32),
            pltpu.VMEM((CB, P), f32),
            pltpu.VMEM((N,), f32),
            pltpu.VMEM((N,), f32),
            pltpu.VMEM((N,), f32),
            pltpu.SemaphoreType.DMA,
            pltpu.SemaphoreType.DMA,
            pltpu.SemaphoreType.DMA,
        ],
    )(_make_edge_gather_body(epw, nchunk))
    return fn(row, col, xa, xb, px, py, pz)


def _edge_mlp_body(xar_ref, xbc_ref, rij_ref, w1ec_ref, W2e_ref,
                   b2e_ref, Wc_ref, bc_ref, m_ref, trans_ref):
    rij = rij_ref[...]
    dij = jnp.sum(rij * rij, axis=1, keepdims=True)
    pre = xar_ref[...] + xbc_ref[...] + dij * w1ec_ref[...]
    h = _silu(pre)
    m = _silu(jnp.dot(h, W2e_ref[...],
                      preferred_element_type=jnp.float32) + b2e_ref[...])
    m_ref[...] = m
    w = _silu(jnp.dot(m, Wc_ref[...],
                      preferred_element_type=jnp.float32) + bc_ref[...])
    rn = rij / (jnp.sqrt(dij) + 1e-8)
    trans_ref[...] = jnp.concatenate(
        [rn * w, jnp.zeros((rij.shape[0], DW - P), jnp.float32)], axis=1)


def _edge_mlp(xar, xbc, rij, w1ec, W2e, b2e, Wc, bc, ne):
    f32 = jnp.float32
    return pl.pallas_call(
        _edge_mlp_body,
        grid=(ne // BE,),
        in_specs=[
            pl.BlockSpec((BE, H), lambda i: (i, 0)),
            pl.BlockSpec((BE, H), lambda i: (i, 0)),
            pl.BlockSpec((BE, P), lambda i: (i, 0)),
            pl.BlockSpec((1, H), lambda i: (0, 0)),
            pl.BlockSpec((H, H), lambda i: (0, 0)),
            pl.BlockSpec((1, H), lambda i: (0, 0)),
            pl.BlockSpec((H, 1), lambda i: (0, 0)),
            pl.BlockSpec((1, 1), lambda i: (0, 0)),
        ],
        out_specs=[
            pl.BlockSpec((BE, H), lambda i: (i, 0)),
            pl.BlockSpec((BE, DW), lambda i: (i, 0)),
        ],
        out_shape=[
            jax.ShapeDtypeStruct((ne, H), f32),
            jax.ShapeDtypeStruct((ne, DW), f32),
        ],
    )(xar, xbc, rij, w1ec, W2e, b2e, Wc, bc)


def _make_scatter_body(width, epw, nchunk):
    def _scatter_body(row_hbm, val_hbm, out_hbm, idxb0, idxb1, vb0, vb1, zb,
                      acc_s, semL):
        cid = lax.axis_index("c")
        sid = lax.axis_index("s")
        wid = sid * NC + cid
        base = wid * epw

        def zrow(r, carry):
            def zcol(j, c2):
                zb[r, pl.ds(j * 16, 16)] = jnp.zeros((16,), jnp.float32)
                return c2
            lax.fori_loop(0, width // 16, zcol, 0)
            return carry

        lax.fori_loop(0, NPW // 5, zrow, 0)
        for k in range(5):
            pltpu.sync_copy(zb, acc_s.at[pl.ds(sid * NPW + k * (NPW // 5),
                                               NPW // 5)])
        plsc.subcore_barrier()

        def issue(ci, vb, idxb):
            pltpu.async_copy(row_hbm.at[pl.ds(base + ci * CB, CB)], idxb,
                             semL)
            pltpu.async_copy(val_hbm.at[pl.ds(base + ci * CB, CB)], vb, semL)

        def process(ci, vb, idxb):
            pltpu.make_async_copy(row_hbm.at[pl.ds(0, CB)], idxb, semL).wait()
            pltpu.make_async_copy(val_hbm.at[pl.ds(0, CB)], vb, semL).wait()
            pltpu.sync_copy(vb, acc_s.at[idxb], add=True)

        issue(0, vb0, idxb0)
        issue(1, vb1, idxb1)

        def pair(i, carry):
            c0 = 2 * i
            process(c0, vb0, idxb0)

            @pl.when(c0 + 2 < nchunk)
            def _():
                issue(c0 + 2, vb0, idxb0)

            process(c0 + 1, vb1, idxb1)

            @pl.when(c0 + 3 < nchunk)
            def _():
                issue(c0 + 3, vb1, idxb1)

            return carry

        lax.fori_loop(0, nchunk // 2, pair, 0)
        process(nchunk - 1, vb0, idxb0)
        plsc.subcore_barrier()

        pltpu.sync_copy(acc_s.at[pl.ds(sid * NPW, NPW)],
                        out_hbm.at[cid, pl.ds(sid * NPW, NPW)])

    return _scatter_body


def _edge_scatter(row, vals, width, ne):
    f32 = jnp.float32
    epw = ne // NW
    nchunk = epw // CB
    mesh = plsc.VectorSubcoreMesh(core_axis_name="c", subcore_axis_name="s",
                                  num_cores=NC, num_subcores=NS)
    fn = functools.partial(
        pl.kernel, mesh=mesh,
        out_type=jax.ShapeDtypeStruct((NC, NPAD, width), f32),
        scratch_types=[
            pltpu.VMEM((CB,), jnp.int32),
            pltpu.VMEM((CB,), jnp.int32),
            pltpu.VMEM((CB, width), f32),
            pltpu.VMEM((CB, width), f32),
            pltpu.VMEM((NPW // 5, width), f32),
            pltpu.VMEM_SHARED((NPAD, width), f32),
            pltpu.SemaphoreType.DMA,
        ],
    )(_make_scatter_body(width, epw, nchunk))
    return fn(row, vals)


def _make_node_mlp_body(n_agg, n_delta):
    def _node_mlp_body(x_ref, xc_ref, posp_ref, W1nb_ref, W2n_ref, b2n_ref,
                       *rest):
        agg_refs = rest[:n_agg]
        delta_refs = rest[n_agg:n_agg + n_delta]
        xnew_ref, posnew_ref = rest[n_agg + n_delta:]
        agg = agg_refs[0][0] + agg_refs[0][1]
        for r in agg_refs[1:]:
            agg = agg + r[0] + r[1]
        hn = _silu(xc_ref[...] + jnp.dot(agg, W1nb_ref[...],
                                         preferred_element_type=jnp.float32))
        xnew_ref[...] = x_ref[...] + jnp.dot(
            hn, W2n_ref[...],
            preferred_element_type=jnp.float32) + b2n_ref[...]
        dsum = delta_refs[0][0] + delta_refs[0][1]
        for r in delta_refs[1:]:
            dsum = dsum + r[0] + r[1]
        posnew_ref[...] = posp_ref[...] + dsum[:, :P]
    return _node_mlp_body


def _node_mlp(x, xc, pos_pad, aggs_list, deltas_list, W1n_b, W2n, b2n):
    f32 = jnp.float32
    part = pl.BlockSpec((NC, BN, H), lambda i: (0, i, 0))
    partd = pl.BlockSpec((NC, BN, DW), lambda i: (0, i, 0))
    return pl.pallas_call(
        _make_node_mlp_body(len(aggs_list), len(deltas_list)),
        grid=(N // BN,),
        in_specs=[
            pl.BlockSpec((BN, D), lambda i: (i, 0)),
            pl.BlockSpec((BN, H), lambda i: (i, 0)),
            pl.BlockSpec((BN, P), lambda i: (i, 0)),
            pl.BlockSpec((H, D), lambda i: (0, 0)),
            pl.BlockSpec((H, D), lambda i: (0, 0)),
            pl.BlockSpec((1, D), lambda i: (0, 0)),
        ] + [part] * len(aggs_list) + [partd] * len(deltas_list),
        out_specs=[
            pl.BlockSpec((BN, D), lambda i: (i, 0)),
            pl.BlockSpec((BN, P), lambda i: (i, 0)),
        ],
        out_shape=[
            jax.ShapeDtypeStruct((N, D), f32),
            jax.ShapeDtypeStruct((N, P), f32),
        ],
    )(x, xc, pos_pad, W1n_b, W2n, b2n, *aggs_list, *deltas_list)


def kernel(x, pos, edge_index, W1e, b1e, W2e, b2e, W1n, b1n, W2n, b2n, Wc, bc):
    f32 = jnp.float32
    ei = edge_index.astype(jnp.int32)
    row = ei[0]
    col = ei[1]
    pos_pad = jnp.pad(pos, ((0, 0), (0, P - 3)))
    W1e_a = W1e[:D]
    W1e_b = W1e[D:2 * D]
    w1ec = W1e[2 * D:2 * D + 1]
    W1n_a = W1n[:D]
    W1n_b = W1n[D:]
    b1e2 = b1e.reshape(1, H)
    b2e2 = b2e.reshape(1, H)
    b1n2 = b1n.reshape(1, H)
    b2n2 = b2n.reshape(1, D)
    bc2 = bc.reshape(1, 1)

    xa, xb, xc = _precompute(x, W1e_a, W1e_b, b1e2, W1n_a, b1n2)
    px, py, pz = pos[:, 0], pos[:, 1], pos[:, 2]
    ns = E // NSLICE
    aggs_list, deltas_list = [], []
    for s in range(NSLICE):
        rs = row[s * ns:(s + 1) * ns]
        cs = col[s * ns:(s + 1) * ns]
        xar, xbc, rij = _edge_gather(rs, cs, xa, xb, px, py, pz, ns)
        m, trans = _edge_mlp(xar, xbc, rij, w1ec, W2e, b2e2, Wc, bc2, ns)
        aggs_list.append(_edge_scatter(rs, m, H, ns))
        deltas_list.append(_edge_scatter(rs, trans, DW, ns))
    x_new, pos_new_pad = _node_mlp(x, xc, pos_pad, aggs_list, deltas_list,
                                   W1n_b, W2n, b2n2)
    return (x_new, pos_new_pad[:, :3])


# speedup vs baseline: 1.0003x; 1.0003x over previous
import functools

import jax
import jax.numpy as jnp
from jax import lax
from jax.experimental import pallas as pl
from jax.experimental.pallas import tpu as pltpu
from jax.experimental.pallas import tpu_sc as plsc

N = 10000
E = 320000
D = 128
H = 128
P = 16

NC = 2
NS = 16
NW = NC * NS
EPW = E // NW
CB = 80
NCHUNK = EPW // CB
NPAD = 10240
NPW = NPAD // NS

BN = 2000
BE = 8000
NSLICE = 1
DW = 128


def _silu(v):
    return v * (1.0 / (1.0 + jnp.exp(-v)))


def _precompute_body(x_ref, wa_ref, wb_ref, b1e_ref, wna_ref,
                     b1n_ref, xa_ref, xb_ref, xc_ref):
    xv = x_ref[...]
    xa_ref[...] = jnp.dot(xv, wa_ref[...],
                          preferred_element_type=jnp.float32) + b1e_ref[...]
    xb_ref[...] = jnp.dot(xv, wb_ref[...], preferred_element_type=jnp.float32)
    xc_ref[...] = jnp.dot(xv, wna_ref[...],
                          preferred_element_type=jnp.float32) + b1n_ref[...]


def _precompute(x, W1e_a, W1e_b, b1e, W1n_a, b1n):
    f32 = jnp.float32
    return pl.pallas_call(
        _precompute_body,
        grid=(N // BN,),
        in_specs=[
            pl.BlockSpec((BN, D), lambda i: (i, 0)),
            pl.BlockSpec((D, H), lambda i: (0, 0)),
            pl.BlockSpec((D, H), lambda i: (0, 0)),
            pl.BlockSpec((1, H), lambda i: (0, 0)),
            pl.BlockSpec((D, H), lambda i: (0, 0)),
            pl.BlockSpec((1, H), lambda i: (0, 0)),
        ],
        out_specs=[
            pl.BlockSpec((BN, H), lambda i: (i, 0)),
            pl.BlockSpec((BN, H), lambda i: (i, 0)),
            pl.BlockSpec((BN, H), lambda i: (i, 0)),
        ],
        out_shape=[
            jax.ShapeDtypeStruct((N, H), f32),
            jax.ShapeDtypeStruct((N, H), f32),
            jax.ShapeDtypeStruct((N, H), f32),
        ],
    )(x, W1e_a, W1e_b, b1e, W1n_a, b1n)


def _make_edge_gather_body(epw, nchunk):
  def _edge_gather_body(row_hbm, col_hbm, xa_hbm, xb_hbm, px_hbm, py_hbm,
                      pz_hbm, xar_hbm, xbc_hbm, rij_hbm,
                      idxr, idxc, bufA0, bufB0, bufA1, bufB1,
                      bufA2, bufB2, bufR, px_v, py_v, pz_v, semG, semW, semR):
    wid = lax.axis_index("s") * NC + lax.axis_index("c")
    base = wid * epw

    pltpu.sync_copy(px_hbm, px_v)
    pltpu.sync_copy(py_hbm, py_v)
    pltpu.sync_copy(pz_hbm, pz_v)
    pltpu.sync_copy(row_hbm.at[pl.ds(base, epw)], idxr)
    pltpu.sync_copy(col_hbm.at[pl.ds(base, epw)], idxc)
    lane = lax.iota(jnp.int32, 16)

    def zr(r, carry):
        bufR[r, pl.ds(0, P)] = jnp.zeros((P,), jnp.float32)
        return carry

    lax.fori_loop(0, CB, zr, 0)

    def issue(ci, bA, bB):
        pltpu.async_copy(xa_hbm.at[idxr.at[pl.ds(ci * CB, CB)]], bA, semG)
        pltpu.async_copy(xb_hbm.at[idxc.at[pl.ds(ci * CB, CB)]], bB, semG)

    def drain_w():
        pltpu.make_async_copy(bufA0, xar_hbm.at[pl.ds(base, CB)], semW).wait()
        pltpu.make_async_copy(bufB0, xbc_hbm.at[pl.ds(base, CB)], semW).wait()

    def drain_r():
        pltpu.make_async_copy(bufR, rij_hbm.at[pl.ds(base, CB)], semR).wait()

    def process(ci, bA, bB):
        @pl.when(ci > 0)
        def _():
            drain_r()

        def sub(k, carry2):
            off = ci * CB + k * 16
            ir = idxr[pl.ds(off, 16)]
            ic = idxc[pl.ds(off, 16)]
            rows = k * 16 + lane
            for c, pv in enumerate((px_v, py_v, pz_v)):
                d = plsc.load_gather(pv, [ir]) - plsc.load_gather(pv, [ic])
                plsc.store_scatter(bufR,
                                   [rows, jnp.full((16,), c, jnp.int32)], d)
            return carry2

        lax.fori_loop(0, CB // 16, sub, 0)
        pltpu.make_async_copy(xa_hbm.at[pl.ds(0, CB)], bA, semG).wait()
        pltpu.make_async_copy(xa_hbm.at[pl.ds(0, CB)], bB, semG).wait()
        cb = base + ci * CB
        pltpu.async_copy(bA, xar_hbm.at[pl.ds(cb, CB)], semW)
        pltpu.async_copy(bB, xbc_hbm.at[pl.ds(cb, CB)], semW)
        pltpu.async_copy(bufR, rij_hbm.at[pl.ds(cb, CB)], semR)

    issue(0, bufA0, bufB0)
    issue(1, bufA1, bufB1)
    issue(2, bufA2, bufB2)

    def triple(i, carry):
        c0 = 3 * i
        sets = ((bufA0, bufB0), (bufA1, bufB1), (bufA2, bufB2))
        for k, (bA, bB) in enumerate(sets):
            process(c0 + k, bA, bB)

            @pl.when(c0 + k + 3 < nchunk)
            def _():
                drain_w()
                issue(c0 + k + 3, bA, bB)

        return carry

    lax.fori_loop(0, nchunk // 3, triple, 0)
    for k in range(nchunk % 3):
        process(nchunk - (nchunk % 3) + k,
                (bufA0, bufA1, bufA2)[k], (bufB0, bufB1, bufB2)[k])
    for _ in range(3):
        drain_w()
    drain_r()
  return _edge_gather_body


def _edge_gather(row, col, xa, xb, px, py, pz, ne):
    f32 = jnp.float32
    epw = ne // NW
    nchunk = epw // CB
    mesh = plsc.VectorSubcoreMesh(core_axis_name="c", subcore_axis_name="s",
                                  num_cores=NC, num_subcores=NS)
    fn = functools.partial(
        pl.kernel, mesh=mesh,
        compiler_params=pltpu.CompilerParams(needs_layout_passes=False),
        out_type=[
            jax.ShapeDtypeStruct((ne, H), f32),
            jax.ShapeDtypeStruct((ne, H), f32),
            jax.ShapeDtypeStruct((ne, P), f32),
        ],
        scratch_types=[
            pltpu.VMEM((epw,), jnp.int32),
            pltpu.VMEM((epw,), jnp.int32),
            pltpu.VMEM((CB, H), f32),
            pltpu.VMEM((CB, H), f32),
            pltpu.VMEM((CB, H), f32),
            pltpu.VMEM((CB, H), f32),
            pltpu.VMEM((CB, H), f32),
            pltpu.VMEM((CB, H), f32),
            pltpu.VMEM((CB, P), f32),
            pltpu.VMEM((N,), f32),
            pltpu.VMEM((N,), f32),
            pltpu.VMEM((N,), f32),
            pltpu.SemaphoreType.DMA,
            pltpu.SemaphoreType.DMA,
            pltpu.SemaphoreType.DMA,
        ],
    )(_make_edge_gather_body(epw, nchunk))
    return fn(row, col, xa, xb, px, py, pz)


def _edge_mlp_body(xar_ref, xbc_ref, rij_ref, w1ec_ref, W2e_ref,
                   b2e_ref, Wc_ref, bc_ref, m_ref, trans_ref):
    rij = rij_ref[...]
    dij = jnp.sum(rij * rij, axis=1, keepdims=True)
    pre = xar_ref[...] + xbc_ref[...] + dij * w1ec_ref[...]
    h = _silu(pre)
    m = _silu(jnp.dot(h, W2e_ref[...],
                      preferred_element_type=jnp.float32) + b2e_ref[...])
    m_ref[...] = m
    w = _silu(jnp.dot(m, Wc_ref[...],
                      preferred_element_type=jnp.float32) + bc_ref[...])
    rn = rij / (jnp.sqrt(dij) + 1e-8)
    trans_ref[...] = jnp.concatenate(
        [rn * w, jnp.zeros((rij.shape[0], DW - P), jnp.float32)], axis=1)


def _edge_mlp(xar, xbc, rij, w1ec, W2e, b2e, Wc, bc, ne):
    f32 = jnp.float32
    return pl.pallas_call(
        _edge_mlp_body,
        grid=(ne // BE,),
        in_specs=[
            pl.BlockSpec((BE, H), lambda i: (i, 0)),
            pl.BlockSpec((BE, H), lambda i: (i, 0)),
            pl.BlockSpec((BE, P), lambda i: (i, 0)),
            pl.BlockSpec((1, H), lambda i: (0, 0)),
            pl.BlockSpec((H, H), lambda i: (0, 0)),
            pl.BlockSpec((1, H), lambda i: (0, 0)),
            pl.BlockSpec((H, 1), lambda i: (0, 0)),
            pl.BlockSpec((1, 1), lambda i: (0, 0)),
        ],
        out_specs=[
            pl.BlockSpec((BE, H), lambda i: (i, 0)),
            pl.BlockSpec((BE, DW), lambda i: (i, 0)),
        ],
        out_shape=[
            jax.ShapeDtypeStruct((ne, H), f32),
            jax.ShapeDtypeStruct((ne, DW), f32),
        ],
    )(xar, xbc, rij, w1ec, W2e, b2e, Wc, bc)


def _make_scatter_body(width, epw, nchunk):
    def _scatter_body(row_hbm, val_hbm, out_hbm, idxb0, idxb1, vb0, vb1, zb,
                      acc_s, semL, semS):
        cid = lax.axis_index("c")
        sid = lax.axis_index("s")
        wid = sid * NC + cid
        base = wid * epw

        def zrow(r, carry):
            def zcol(j, c2):
                zb[r, pl.ds(j * 16, 16)] = jnp.zeros((16,), jnp.float32)
                return c2
            lax.fori_loop(0, width // 16, zcol, 0)
            return carry

        lax.fori_loop(0, NPW // 5, zrow, 0)
        for k in range(5):
            pltpu.sync_copy(zb, acc_s.at[pl.ds(sid * NPW + k * (NPW // 5),
                                               NPW // 5)])
        plsc.subcore_barrier()

        def issue(ci, vb, idxb):
            pltpu.async_copy(row_hbm.at[pl.ds(base + ci * CB, CB)], idxb,
                             semL)
            pltpu.async_copy(val_hbm.at[pl.ds(base + ci * CB, CB)], vb, semL)

        def drain_s():
            pltpu.make_async_copy(vb0, acc_s.at[pl.ds(0, CB)], semS).wait()

        def process(ci, vb, idxb):
            pltpu.make_async_copy(row_hbm.at[pl.ds(0, CB)], idxb, semL).wait()
            pltpu.make_async_copy(val_hbm.at[pl.ds(0, CB)], vb, semL).wait()
            pltpu.async_copy(vb, acc_s.at[idxb], semS, add=True)

        issue(0, vb0, idxb0)
        issue(1, vb1, idxb1)

        def pair(i, carry):
            c0 = 2 * i
            process(c0, vb0, idxb0)

            @pl.when(c0 + 2 < nchunk)
            def _():
                drain_s()
                issue(c0 + 2, vb0, idxb0)

            process(c0 + 1, vb1, idxb1)

            @pl.when(c0 + 3 < nchunk)
            def _():
                drain_s()
                issue(c0 + 3, vb1, idxb1)

            return carry

        lax.fori_loop(0, nchunk // 2, pair, 0)
        process(nchunk - 1, vb0, idxb0)
        drain_s()
        drain_s()
        plsc.subcore_barrier()

        pltpu.sync_copy(acc_s.at[pl.ds(sid * NPW, NPW)],
                        out_hbm.at[cid, pl.ds(sid * NPW, NPW)])

    return _scatter_body


def _edge_scatter(row, vals, width, ne):
    f32 = jnp.float32
    epw = ne // NW
    nchunk = epw // CB
    mesh = plsc.VectorSubcoreMesh(core_axis_name="c", subcore_axis_name="s",
                                  num_cores=NC, num_subcores=NS)
    fn = functools.partial(
        pl.kernel, mesh=mesh,
        out_type=jax.ShapeDtypeStruct((NC, NPAD, width), f32),
        scratch_types=[
            pltpu.VMEM((CB,), jnp.int32),
            pltpu.VMEM((CB,), jnp.int32),
            pltpu.VMEM((CB, width), f32),
            pltpu.VMEM((CB, width), f32),
            pltpu.VMEM((NPW // 5, width), f32),
            pltpu.VMEM_SHARED((NPAD, width), f32),
            pltpu.SemaphoreType.DMA,
            pltpu.SemaphoreType.DMA,
        ],
    )(_make_scatter_body(width, epw, nchunk))
    return fn(row, vals)


def _make_node_mlp_body(n_agg, n_delta):
    def _node_mlp_body(x_ref, xc_ref, posp_ref, W1nb_ref, W2n_ref, b2n_ref,
                       *rest):
        agg_refs = rest[:n_agg]
        delta_refs = rest[n_agg:n_agg + n_delta]
        xnew_ref, posnew_ref = rest[n_agg + n_delta:]
        agg = agg_refs[0][0] + agg_refs[0][1]
        for r in agg_refs[1:]:
            agg = agg + r[0] + r[1]
        hn = _silu(xc_ref[...] + jnp.dot(agg, W1nb_ref[...],
                                         preferred_element_type=jnp.float32))
        xnew_ref[...] = x_ref[...] + jnp.dot(
            hn, W2n_ref[...],
            preferred_element_type=jnp.float32) + b2n_ref[...]
        dsum = delta_refs[0][0] + delta_refs[0][1]
        for r in delta_refs[1:]:
            dsum = dsum + r[0] + r[1]
        posnew_ref[...] = posp_ref[...] + dsum[:, :P]
    return _node_mlp_body


def _node_mlp(x, xc, pos_pad, aggs_list, deltas_list, W1n_b, W2n, b2n):
    f32 = jnp.float32
    part = pl.BlockSpec((NC, BN, H), lambda i: (0, i, 0))
    partd = pl.BlockSpec((NC, BN, DW), lambda i: (0, i, 0))
    return pl.pallas_call(
        _make_node_mlp_body(len(aggs_list), len(deltas_list)),
        grid=(N // BN,),
        in_specs=[
            pl.BlockSpec((BN, D), lambda i: (i, 0)),
            pl.BlockSpec((BN, H), lambda i: (i, 0)),
            pl.BlockSpec((BN, P), lambda i: (i, 0)),
            pl.BlockSpec((H, D), lambda i: (0, 0)),
            pl.BlockSpec((H, D), lambda i: (0, 0)),
            pl.BlockSpec((1, D), lambda i: (0, 0)),
        ] + [part] * len(aggs_list) + [partd] * len(deltas_list),
        out_specs=[
            pl.BlockSpec((BN, D), lambda i: (i, 0)),
            pl.BlockSpec((BN, P), lambda i: (i, 0)),
        ],
        out_shape=[
            jax.ShapeDtypeStruct((N, D), f32),
            jax.ShapeDtypeStruct((N, P), f32),
        ],
    )(x, xc, pos_pad, W1n_b, W2n, b2n, *aggs_list, *deltas_list)


def kernel(x, pos, edge_index, W1e, b1e, W2e, b2e, W1n, b1n, W2n, b2n, Wc, bc):
    f32 = jnp.float32
    ei = edge_index.astype(jnp.int32)
    row = ei[0]
    col = ei[1]
    pos_pad = jnp.pad(pos, ((0, 0), (0, P - 3)))
    W1e_a = W1e[:D]
    W1e_b = W1e[D:2 * D]
    w1ec = W1e[2 * D:2 * D + 1]
    W1n_a = W1n[:D]
    W1n_b = W1n[D:]
    b1e2 = b1e.reshape(1, H)
    b2e2 = b2e.reshape(1, H)
    b1n2 = b1n.reshape(1, H)
    b2n2 = b2n.reshape(1, D)
    bc2 = bc.reshape(1, 1)

    xa, xb, xc = _precompute(x, W1e_a, W1e_b, b1e2, W1n_a, b1n2)
    px, py, pz = pos[:, 0], pos[:, 1], pos[:, 2]
    ns = E // NSLICE
    aggs_list, deltas_list = [], []
    for s in range(NSLICE):
        rs = row[s * ns:(s + 1) * ns]
        cs = col[s * ns:(s + 1) * ns]
        xar, xbc, rij = _edge_gather(rs, cs, xa, xb, px, py, pz, ns)
        m, trans = _edge_mlp(xar, xbc, rij, w1ec, W2e, b2e2, Wc, bc2, ns)
        aggs_list.append(_edge_scatter(rs, m, H, ns))
        deltas_list.append(_edge_scatter(rs, trans, DW, ns))
    x_new, pos_new_pad = _node_mlp(x, xc, pos_pad, aggs_list, deltas_list,
                                   W1n_b, W2n, b2n2)
    return (x_new, pos_new_pad[:, :3])


# speedup vs baseline: 1.0042x; 1.0040x over previous
import functools

import jax
import jax.numpy as jnp
from jax import lax
from jax.experimental import pallas as pl
from jax.experimental.pallas import tpu as pltpu
from jax.experimental.pallas import tpu_sc as plsc

N = 10000
E = 320000
D = 128
H = 128
P = 16

NC = 2
NS = 16
NW = NC * NS
EPW = E // NW
CB = 80
NCHUNK = EPW // CB
NPAD = 10240
NPW = NPAD // NS

BN = 2000
BE = 8000
NSLICE = 1
DW = 128


def _silu(v):
    return v * (1.0 / (1.0 + jnp.exp(-v)))


def _precompute_body(x_ref, wa_ref, wb_ref, b1e_ref, wna_ref,
                     b1n_ref, xa_ref, xb_ref, xc_ref):
    xv = x_ref[...]
    xa_ref[...] = jnp.dot(xv, wa_ref[...],
                          preferred_element_type=jnp.float32) + b1e_ref[...]
    xb_ref[...] = jnp.dot(xv, wb_ref[...], preferred_element_type=jnp.float32)
    xc_ref[...] = jnp.dot(xv, wna_ref[...],
                          preferred_element_type=jnp.float32) + b1n_ref[...]


def _precompute(x, W1e_a, W1e_b, b1e, W1n_a, b1n):
    f32 = jnp.float32
    return pl.pallas_call(
        _precompute_body,
        grid=(N // BN,),
        in_specs=[
            pl.BlockSpec((BN, D), lambda i: (i, 0)),
            pl.BlockSpec((D, H), lambda i: (0, 0)),
            pl.BlockSpec((D, H), lambda i: (0, 0)),
            pl.BlockSpec((1, H), lambda i: (0, 0)),
            pl.BlockSpec((D, H), lambda i: (0, 0)),
            pl.BlockSpec((1, H), lambda i: (0, 0)),
        ],
        out_specs=[
            pl.BlockSpec((BN, H), lambda i: (i, 0)),
            pl.BlockSpec((BN, H), lambda i: (i, 0)),
            pl.BlockSpec((BN, H), lambda i: (i, 0)),
        ],
        out_shape=[
            jax.ShapeDtypeStruct((N, H), f32),
            jax.ShapeDtypeStruct((N, H), f32),
            jax.ShapeDtypeStruct((N, H), f32),
        ],
    )(x, W1e_a, W1e_b, b1e, W1n_a, b1n)


def _make_edge_gather_body(epw, nchunk):
  def _edge_gather_body(row_hbm, col_hbm, xa_hbm, xb_hbm, px_hbm, py_hbm,
                      pz_hbm, xar_hbm, xbc_hbm, rij_hbm,
                      idxr, idxc, bufA0, bufB0, bufA1, bufB1,
                      bufA2, bufB2, bufR, px_v, py_v, pz_v, semG, semW, semR):
    wid = lax.axis_index("s") * NC + lax.axis_index("c")
    base = wid * epw

    pltpu.sync_copy(px_hbm, px_v)
    pltpu.sync_copy(py_hbm, py_v)
    pltpu.sync_copy(pz_hbm, pz_v)
    pltpu.sync_copy(row_hbm.at[pl.ds(base, epw)], idxr)
    pltpu.sync_copy(col_hbm.at[pl.ds(base, epw)], idxc)
    lane = lax.iota(jnp.int32, 16)

    def zr(r, carry):
        bufR[r, pl.ds(0, P)] = jnp.zeros((P,), jnp.float32)
        return carry

    lax.fori_loop(0, CB, zr, 0)

    def issue(ci, bA, bB):
        pltpu.async_copy(xa_hbm.at[idxr.at[pl.ds(ci * CB, CB)]], bA, semG)
        pltpu.async_copy(xb_hbm.at[idxc.at[pl.ds(ci * CB, CB)]], bB, semG)

    def drain_w():
        pltpu.make_async_copy(bufA0, xar_hbm.at[pl.ds(base, CB)], semW).wait()
        pltpu.make_async_copy(bufB0, xbc_hbm.at[pl.ds(base, CB)], semW).wait()

    def drain_r():
        pltpu.make_async_copy(bufR, rij_hbm.at[pl.ds(base, CB)], semR).wait()

    def process(ci, bA, bB):
        @pl.when(ci > 0)
        def _():
            drain_r()

        def sub(k, carry2):
            off = ci * CB + k * 16
            ir = idxr[pl.ds(off, 16)]
            ic = idxc[pl.ds(off, 16)]
            rows = k * 16 + lane
            for c, pv in enumerate((px_v, py_v, pz_v)):
                d = plsc.load_gather(pv, [ir]) - plsc.load_gather(pv, [ic])
                plsc.store_scatter(bufR,
                                   [rows, jnp.full((16,), c, jnp.int32)], d)
            return carry2

        lax.fori_loop(0, CB // 16, sub, 0)
        pltpu.make_async_copy(xa_hbm.at[pl.ds(0, CB)], bA, semG).wait()
        pltpu.make_async_copy(xa_hbm.at[pl.ds(0, CB)], bB, semG).wait()
        cb = base + ci * CB
        pltpu.async_copy(bA, xar_hbm.at[pl.ds(cb, CB)], semW)
        pltpu.async_copy(bB, xbc_hbm.at[pl.ds(cb, CB)], semW)
        pltpu.async_copy(bufR, rij_hbm.at[pl.ds(cb, CB)], semR)

    issue(0, bufA0, bufB0)
    issue(1, bufA1, bufB1)
    issue(2, bufA2, bufB2)

    def triple(i, carry):
        c0 = 3 * i
        sets = ((bufA0, bufB0), (bufA1, bufB1), (bufA2, bufB2))
        for k, (bA, bB) in enumerate(sets):
            process(c0 + k, bA, bB)

            @pl.when(c0 + k + 3 < nchunk)
            def _():
                drain_w()
                issue(c0 + k + 3, bA, bB)

        return carry

    lax.fori_loop(0, nchunk // 3, triple, 0)
    for k in range(nchunk % 3):
        process(nchunk - (nchunk % 3) + k,
                (bufA0, bufA1, bufA2)[k], (bufB0, bufB1, bufB2)[k])
    for _ in range(3):
        drain_w()
    drain_r()
  return _edge_gather_body


def _edge_gather(row, col, xa, xb, px, py, pz, ne):
    f32 = jnp.float32
    epw = ne // NW
    nchunk = epw // CB
    mesh = plsc.VectorSubcoreMesh(core_axis_name="c", subcore_axis_name="s",
                                  num_cores=NC, num_subcores=NS)
    fn = functools.partial(
        pl.kernel, mesh=mesh,
        compiler_params=pltpu.CompilerParams(needs_layout_passes=False),
        out_type=[
            jax.ShapeDtypeStruct((ne, H), f32),
            jax.ShapeDtypeStruct((ne, H), f32),
            jax.ShapeDtypeStruct((ne, P), f32),
        ],
        scratch_types=[
            pltpu.VMEM((epw,), jnp.int32),
            pltpu.VMEM((epw,), jnp.int32),
            pltpu.VMEM((CB, H), f32),
            pltpu.VMEM((CB, H), f32),
            pltpu.VMEM((CB, H), f32),
            pltpu.VMEM((CB, H), f32),
            pltpu.VMEM((CB, H), f32),
            pltpu.VMEM((CB, H), f32),
            pltpu.VMEM((CB, P), f32),
            pltpu.VMEM((N,), f32),
            pltpu.VMEM((N,), f32),
            pltpu.VMEM((N,), f32),
            pltpu.SemaphoreType.DMA,
            pltpu.SemaphoreType.DMA,
            pltpu.SemaphoreType.DMA,
        ],
    )(_make_edge_gather_body(epw, nchunk))
    return fn(row, col, xa, xb, px, py, pz)


def _edge_mlp_body(xar_ref, xbc_ref, rij_ref, w1ec_ref, W2e_ref,
                   b2e_ref, Wc_ref, bc_ref, m_ref, trans_ref):
    rij = rij_ref[...]
    dij = jnp.sum(rij * rij, axis=1, keepdims=True)
    pre = xar_ref[...] + xbc_ref[...] + dij * w1ec_ref[...]
    h = _silu(pre)
    m = _silu(jnp.dot(h, W2e_ref[...],
                      preferred_element_type=jnp.float32) + b2e_ref[...])
    m_ref[...] = m
    w = _silu(jnp.dot(m, Wc_ref[...],
                      preferred_element_type=jnp.float32) + bc_ref[...])
    rn = rij / (jnp.sqrt(dij) + 1e-8)
    trans_ref[...] = jnp.concatenate(
        [rn * w, jnp.zeros((rij.shape[0], DW - P), jnp.float32)], axis=1)


def _edge_mlp(xar, xbc, rij, w1ec, W2e, b2e, Wc, bc, ne):
    f32 = jnp.float32
    return pl.pallas_call(
        _edge_mlp_body,
        grid=(ne // BE,),
        in_specs=[
            pl.BlockSpec((BE, H), lambda i: (i, 0)),
            pl.BlockSpec((BE, H), lambda i: (i, 0)),
            pl.BlockSpec((BE, P), lambda i: (i, 0)),
            pl.BlockSpec((1, H), lambda i: (0, 0)),
            pl.BlockSpec((H, H), lambda i: (0, 0)),
            pl.BlockSpec((1, H), lambda i: (0, 0)),
            pl.BlockSpec((H, 1), lambda i: (0, 0)),
            pl.BlockSpec((1, 1), lambda i: (0, 0)),
        ],
        out_specs=[
            pl.BlockSpec((BE, H), lambda i: (i, 0)),
            pl.BlockSpec((BE, DW), lambda i: (i, 0)),
        ],
        out_shape=[
            jax.ShapeDtypeStruct((ne, H), f32),
            jax.ShapeDtypeStruct((ne, DW), f32),
        ],
    )(xar, xbc, rij, w1ec, W2e, b2e, Wc, bc)


def _make_scatter_body(width, epw, nchunk):
    def _scatter_body(row_hbm, val_hbm, out_hbm, idxb0, idxb1, vb0, vb1, zb,
                      acc_s, semL, semS):
        cid = lax.axis_index("c")
        sid = lax.axis_index("s")
        wid = sid * NC + cid
        base = wid * epw

        def zrow(r, carry):
            def zcol(j, c2):
                zb[r, pl.ds(j * 16, 16)] = jnp.zeros((16,), jnp.float32)
                return c2
            lax.fori_loop(0, width // 16, zcol, 0)
            return carry

        lax.fori_loop(0, NPW // 5, zrow, 0)
        for k in range(5):
            pltpu.async_copy(zb, acc_s.at[pl.ds(sid * NPW + k * (NPW // 5),
                                                NPW // 5)], semS)

        def issue(ci, vb, idxb):
            pltpu.async_copy(row_hbm.at[pl.ds(base + ci * CB, CB)], idxb,
                             semL)
            pltpu.async_copy(val_hbm.at[pl.ds(base + ci * CB, CB)], vb, semL)

        def drain_s():
            pltpu.make_async_copy(vb0, acc_s.at[pl.ds(0, CB)], semS).wait()

        def process(ci, vb, idxb):
            pltpu.make_async_copy(row_hbm.at[pl.ds(0, CB)], idxb, semL).wait()
            pltpu.make_async_copy(val_hbm.at[pl.ds(0, CB)], vb, semL).wait()
            pltpu.async_copy(vb, acc_s.at[idxb], semS, add=True)

        issue(0, vb0, idxb0)
        issue(1, vb1, idxb1)
        for k in range(5):
            pltpu.make_async_copy(zb, acc_s.at[pl.ds(0, NPW // 5)],
                                  semS).wait()
        plsc.subcore_barrier()

        def pair(i, carry):
            c0 = 2 * i
            process(c0, vb0, idxb0)

            @pl.when(c0 + 2 < nchunk)
            def _():
                drain_s()
                issue(c0 + 2, vb0, idxb0)

            process(c0 + 1, vb1, idxb1)

            @pl.when(c0 + 3 < nchunk)
            def _():
                drain_s()
                issue(c0 + 3, vb1, idxb1)

            return carry

        lax.fori_loop(0, nchunk // 2, pair, 0)
        process(nchunk - 1, vb0, idxb0)
        drain_s()
        drain_s()
        plsc.subcore_barrier()

        pltpu.sync_copy(acc_s.at[pl.ds(sid * NPW, NPW)],
                        out_hbm.at[cid, pl.ds(sid * NPW, NPW)])

    return _scatter_body


def _edge_scatter(row, vals, width, ne):
    f32 = jnp.float32
    epw = ne // NW
    nchunk = epw // CB
    mesh = plsc.VectorSubcoreMesh(core_axis_name="c", subcore_axis_name="s",
                                  num_cores=NC, num_subcores=NS)
    fn = functools.partial(
        pl.kernel, mesh=mesh,
        out_type=jax.ShapeDtypeStruct((NC, NPAD, width), f32),
        scratch_types=[
            pltpu.VMEM((CB,), jnp.int32),
            pltpu.VMEM((CB,), jnp.int32),
            pltpu.VMEM((CB, width), f32),
            pltpu.VMEM((CB, width), f32),
            pltpu.VMEM((NPW // 5, width), f32),
            pltpu.VMEM_SHARED((NPAD, width), f32),
            pltpu.SemaphoreType.DMA,
            pltpu.SemaphoreType.DMA,
        ],
    )(_make_scatter_body(width, epw, nchunk))
    return fn(row, vals)


def _make_node_mlp_body(n_agg, n_delta):
    def _node_mlp_body(x_ref, xc_ref, posp_ref, W1nb_ref, W2n_ref, b2n_ref,
                       *rest):
        agg_refs = rest[:n_agg]
        delta_refs = rest[n_agg:n_agg + n_delta]
        xnew_ref, posnew_ref = rest[n_agg + n_delta:]
        agg = agg_refs[0][0] + agg_refs[0][1]
        for r in agg_refs[1:]:
            agg = agg + r[0] + r[1]
        hn = _silu(xc_ref[...] + jnp.dot(agg, W1nb_ref[...],
                                         preferred_element_type=jnp.float32))
        xnew_ref[...] = x_ref[...] + jnp.dot(
            hn, W2n_ref[...],
            preferred_element_type=jnp.float32) + b2n_ref[...]
        dsum = delta_refs[0][0] + delta_refs[0][1]
        for r in delta_refs[1:]:
            dsum = dsum + r[0] + r[1]
        posnew_ref[...] = posp_ref[...] + dsum[:, :P]
    return _node_mlp_body


def _node_mlp(x, xc, pos_pad, aggs_list, deltas_list, W1n_b, W2n, b2n):
    f32 = jnp.float32
    part = pl.BlockSpec((NC, BN, H), lambda i: (0, i, 0))
    partd = pl.BlockSpec((NC, BN, DW), lambda i: (0, i, 0))
    return pl.pallas_call(
        _make_node_mlp_body(len(aggs_list), len(deltas_list)),
        grid=(N // BN,),
        in_specs=[
            pl.BlockSpec((BN, D), lambda i: (i, 0)),
            pl.BlockSpec((BN, H), lambda i: (i, 0)),
            pl.BlockSpec((BN, P), lambda i: (i, 0)),
            pl.BlockSpec((H, D), lambda i: (0, 0)),
            pl.BlockSpec((H, D), lambda i: (0, 0)),
            pl.BlockSpec((1, D), lambda i: (0, 0)),
        ] + [part] * len(aggs_list) + [partd] * len(deltas_list),
        out_specs=[
            pl.BlockSpec((BN, D), lambda i: (i, 0)),
            pl.BlockSpec((BN, P), lambda i: (i, 0)),
        ],
        out_shape=[
            jax.ShapeDtypeStruct((N, D), f32),
            jax.ShapeDtypeStruct((N, P), f32),
        ],
    )(x, xc, pos_pad, W1n_b, W2n, b2n, *aggs_list, *deltas_list)


def kernel(x, pos, edge_index, W1e, b1e, W2e, b2e, W1n, b1n, W2n, b2n, Wc, bc):
    f32 = jnp.float32
    ei = edge_index.astype(jnp.int32)
    row = ei[0]
    col = ei[1]
    pos_pad = jnp.pad(pos, ((0, 0), (0, P - 3)))
    W1e_a = W1e[:D]
    W1e_b = W1e[D:2 * D]
    w1ec = W1e[2 * D:2 * D + 1]
    W1n_a = W1n[:D]
    W1n_b = W1n[D:]
    b1e2 = b1e.reshape(1, H)
    b2e2 = b2e.reshape(1, H)
    b1n2 = b1n.reshape(1, H)
    b2n2 = b2n.reshape(1, D)
    bc2 = bc.reshape(1, 1)

    xa, xb, xc = _precompute(x, W1e_a, W1e_b, b1e2, W1n_a, b1n2)
    px, py, pz = pos[:, 0], pos[:, 1], pos[:, 2]
    ns = E // NSLICE
    aggs_list, deltas_list = [], []
    for s in range(NSLICE):
        rs = row[s * ns:(s + 1) * ns]
        cs = col[s * ns:(s + 1) * ns]
        xar, xbc, rij = _edge_gather(rs, cs, xa, xb, px, py, pz, ns)
        m, trans = _edge_mlp(xar, xbc, rij, w1ec, W2e, b2e2, Wc, bc2, ns)
        aggs_list.append(_edge_scatter(rs, m, H, ns))
        deltas_list.append(_edge_scatter(rs, trans, DW, ns))
    x_new, pos_new_pad = _node_mlp(x, xc, pos_pad, aggs_list, deltas_list,
                                   W1n_b, W2n, b2n2)
    return (x_new, pos_new_pad[:, :3])
